# baseline (speedup 1.0000x reference)
import jax
import jax.numpy as jnp
from jax import lax
from jax.experimental import pallas as pl
from jax.experimental.pallas import tpu as pltpu
from jax.experimental.pallas import tpu_sc as plsc

N_ATOMS = 16384
N_GRAPHS = 16
TOP_B = 5
NS = 16
CHUNK = N_ATOMS // NS
LANES = 16


def _sc_body(batch_hbm, aw_hbm, si_hbm, sw_hbm,
             batch_v, hist_v, allhist_v, inv_v, out_v, sel_i_v, sel_w_v,
             shared_hist):
    s = lax.axis_index("s")
    base = s * CHUNK

    pltpu.sync_copy(batch_hbm.at[pl.ds(base, CHUNK)], batch_v)

    hist_v[...] = jnp.zeros((LANES,), jnp.int32)
    ones = jnp.ones((LANES,), jnp.int32)
    for i in range(CHUNK // LANES):
        idx = batch_v[pl.ds(i * LANES, LANES)]
        plsc.addupdate_scatter(hist_v, [idx], ones)

    pltpu.sync_copy(hist_v, shared_hist.at[pl.ds(s * LANES, LANES)])
    plsc.subcore_barrier()
    pltpu.sync_copy(shared_hist, allhist_v)

    counts = allhist_v[pl.ds(0, LANES)]
    for i in range(1, NS):
        counts = counts + allhist_v[pl.ds(i * LANES, LANES)]
    inv = 1.0 / counts.astype(jnp.float32)
    inv_v[...] = inv

    for i in range(CHUNK // LANES):
        idx = batch_v[pl.ds(i * LANES, LANES)]
        out_v[pl.ds(i * LANES, LANES)] = plsc.load_gather(inv_v, [idx])
    pltpu.sync_copy(out_v, aw_hbm.at[pl.ds(base, CHUNK)])

    @pl.when(s == 0)
    def _():
        starts = plsc.cumsum(counts) - counts
        gid = lax.iota(jnp.int32, LANES)
        for j in range(TOP_B):
            pos = gid * TOP_B + j
            plsc.store_scatter(sel_i_v, [pos], starts + j)
            plsc.store_scatter(sel_w_v, [pos], inv)
        pltpu.sync_copy(sel_i_v, si_hbm)
        pltpu.sync_copy(sel_w_v, sw_hbm)


_sc_call = pl.kernel(
    _sc_body,
    out_type=(
        jax.ShapeDtypeStruct((N_ATOMS,), jnp.float32),
        jax.ShapeDtypeStruct((N_GRAPHS * TOP_B,), jnp.int32),
        jax.ShapeDtypeStruct((N_GRAPHS * TOP_B,), jnp.float32),
    ),
    mesh=plsc.VectorSubcoreMesh(
        core_axis_name="c", subcore_axis_name="s", num_cores=1,
        num_subcores=NS),
    compiler_params=pltpu.CompilerParams(needs_layout_passes=False),
    scratch_types=(
        pltpu.VMEM((CHUNK,), jnp.int32),
        pltpu.VMEM((LANES,), jnp.int32),
        pltpu.VMEM((NS * LANES,), jnp.int32),
        pltpu.VMEM((LANES,), jnp.float32),
        pltpu.VMEM((CHUNK,), jnp.float32),
        pltpu.VMEM((N_GRAPHS * TOP_B,), jnp.int32),
        pltpu.VMEM((N_GRAPHS * TOP_B,), jnp.float32),
        pltpu.VMEM_SHARED((NS * LANES,), jnp.int32),
    ),
)


@jax.jit
def kernel(node_repr, graph_repr, prototypes, batch):
    atom_weights, selected_indices, selected_weights = _sc_call(
        batch.astype(jnp.int32))
    return (
        graph_repr,
        atom_weights,
        selected_indices.astype(batch.dtype),
        selected_weights,
    )
